# baseline (speedup 1.0000x reference)
import functools

import jax
import jax.numpy as jnp
from jax import lax
from jax.experimental import pallas as pl
from jax.experimental.pallas import tpu as pltpu
from jax.experimental.pallas import tpu_sc as plsc

NC = 2
NS = 16
TILES = NC * NS
CH = 88

F32 = jnp.float32
I32 = jnp.int32


def _sc_mesh():
    return plsc.VectorSubcoreMesh(core_axis_name="c", subcore_axis_name="s")


def _make_scatter_kernel(NPA, D, K0, K1, KM):
    rows_per_tile = NPA // NS

    NBUF = 4
    NI = 6
    S = 2

    def body(g_hbm, idx_hbm, out_hbm, idx_v, rows_v, acc_sh, gsem, ssem,
             isem):
        cid = lax.axis_index("c")
        sid = lax.axis_index("s")
        tid = cid * NS + sid
        KT = jnp.where(cid == 0, K0, K1)
        zeros = jnp.zeros((16,), F32)

        def zbody(i, _):
            for l in range(8):
                rows_v[0, i, pl.ds(l * 16, 16)] = zeros
            return _

        lax.fori_loop(0, CH, zbody, None)
        base = sid * rows_per_tile
        nfull = rows_per_tile // CH
        for r in range(nfull):
            pltpu.sync_copy(rows_v.at[0], acc_sh.at[pl.ds(base + r * CH, CH)])
        rem = rows_per_tile - nfull * CH
        if rem:
            pltpu.sync_copy(rows_v.at[0, pl.ds(0, rem)],
                            acc_sh.at[pl.ds(base + nfull * CH, rem)])
        plsc.subcore_barrier()

        def idx_load(c):
            pltpu.async_copy(idx_hbm.at[tid, c], idx_v.at[lax.rem(c, NI)],
                             isem)

        def idx_wait():
            pltpu.make_async_copy(idx_hbm.at[tid, 0], idx_v.at[0],
                                  isem).wait()

        def gather(c):
            pltpu.async_copy(g_hbm.at[idx_v.at[lax.rem(c, NI), 0]],
                             rows_v.at[lax.rem(c, NBUF)], gsem)

        for c in range(NI):
            idx_load(c)
        for c in range(NBUF):
            idx_wait()
            gather(c)

        def chunk(j, _):
            b = lax.rem(j, NBUF)
            pltpu.make_async_copy(
                g_hbm.at[idx_v.at[0, 0]], rows_v.at[b], gsem).wait()
            pltpu.async_copy(rows_v.at[b],
                             acc_sh.at[idx_v.at[lax.rem(j, NI), 1]],
                             ssem, add=True)

            @pl.when(j >= S)
            def _():
                jd = j - S
                pltpu.make_async_copy(
                    rows_v.at[lax.rem(jd, NBUF)], acc_sh.at[idx_v.at[0, 1]],
                    ssem).wait()

                @pl.when(jd + NI < KT)
                def _():
                    idx_load(jd + NI)

                @pl.when(jd + NBUF < KT)
                def _():
                    idx_wait()
                    gather(jd + NBUF)

            return _

        lax.fori_loop(0, KT, chunk, None)
        for t in range(S):
            pltpu.make_async_copy(
                rows_v.at[t], acc_sh.at[idx_v.at[0, 1]], ssem).wait()

        plsc.subcore_barrier()
        sl = pl.ds(base, rows_per_tile)
        pltpu.sync_copy(acc_sh.at[sl], out_hbm.at[cid, sl])

    return pl.kernel(
        body,
        out_type=jax.ShapeDtypeStruct((NC, NPA, D), F32),
        mesh=_sc_mesh(),
        scratch_types=[
            pltpu.VMEM((NI, 2, CH), I32),
            pltpu.VMEM((NBUF, CH, D), F32),
            pltpu.VMEM_SHARED((NPA, D), F32),
            pltpu.SemaphoreType.DMA,
            pltpu.SemaphoreType.DMA,
            pltpu.SemaphoreType.DMA,
        ],
    )


def _dinv_block(degp, b, i, R, NN, D):
    deg = degp[0, :, 0] + degp[1, :, 0]
    rowid = lax.broadcasted_iota(I32, (R,), 0) + i * R
    valid = (deg > 0) & (rowid < NN)
    return jnp.where(valid, lax.rsqrt(jnp.maximum(deg, 1.0)), 0.0)


def _make_tc_pre(NP, D, NN, R):
    grid = NP // R

    def body(x1_ref, x2_ref, degp1_ref, degp2_ref, w1_ref, b1_ref,
             w2_ref, b2_ref, h1_ref, g1_ref, h2_ref, g2_ref):
        i = pl.program_id(0)
        for b, (x_ref, degp_ref, w_ref, bb_ref, h_ref, g_ref) in enumerate(
                ((x1_ref, degp1_ref, w1_ref, b1_ref, h1_ref, g1_ref),
                 (x2_ref, degp2_ref, w2_ref, b2_ref, h2_ref, g2_ref))):
            h = jnp.maximum(
                jnp.dot(x_ref[...], w_ref[...],
                        preferred_element_type=F32) + bb_ref[...], 0.0)
            dinv = _dinv_block(degp_ref, b, i, R, NN, D)
            h_ref[...] = h
            g_ref[...] = h * dinv[:, None]

    blk = pl.BlockSpec((R, D), lambda i: (i, 0))
    wspec = pl.BlockSpec((D, D), lambda i: (0, 0))
    bspec = pl.BlockSpec((1, D), lambda i: (0, 0))
    return pl.pallas_call(
        body,
        grid=(grid,),
        in_specs=[blk, blk,
                  pl.BlockSpec((NC, R, D), lambda i: (0, i, 0)),
                  pl.BlockSpec((NC, R, D), lambda i: (0, i, 0)),
                  wspec, bspec, wspec, bspec],
        out_specs=[blk, blk, blk, blk],
        out_shape=[jax.ShapeDtypeStruct((NP, D), F32)] * 4,
    )


def _make_tc_post(NP, D, NN, G, R):
    grid = NP // R

    def body(h1_ref, tx1_ref, bat1_ref, w01_ref, w11_ref, bb1_ref,
             h2_ref, tx2_ref, bat2_ref, w02_ref, w12_ref, bb2_ref,
             degp1_ref, degp2_ref, f1_ref, f2_ref, fb_ref,
             pred_ref, p1_ref, p2_ref):
        i = pl.program_id(0)

        @pl.when(i == 0)
        def _():
            p1_ref[...] = jnp.zeros((G, D), F32)
            p2_ref[...] = jnp.zeros((G, D), F32)

        for b, (h_ref, tx_ref, bat_ref, degp_ref, w0_ref, w1_ref, bb_ref,
                p_ref) in \
                enumerate(((h1_ref, tx1_ref, bat1_ref, degp1_ref, w01_ref,
                            w11_ref, bb1_ref, p1_ref),
                           (h2_ref, tx2_ref, bat2_ref, degp2_ref, w02_ref,
                            w12_ref, bb2_ref, p2_ref))):
            dinv = _dinv_block(degp_ref, b, i, R, NN, D)
            t = (tx_ref[0] + tx_ref[1]) * (-dinv)[:, None]
            out = jnp.maximum(
                jnp.dot(h_ref[...], w0_ref[...], preferred_element_type=F32)
                + jnp.dot(t, w1_ref[...], preferred_element_type=F32)
                + bb_ref[...], 0.0)
            bat = bat_ref[0, 0, :]
            mask = (bat[:, None] ==
                    lax.broadcasted_iota(I32, (1, G), 1)).astype(F32)
            p_ref[...] += lax.dot_general(
                mask, out, (((0,), (0,)), ((), ())),
                preferred_element_type=F32)

        pred_ref[...] = (
            jnp.dot(p1_ref[...], f1_ref[...], preferred_element_type=F32)
            + jnp.dot(p2_ref[...], f2_ref[...], preferred_element_type=F32)
            + fb_ref[...])

    blk = pl.BlockSpec((R, D), lambda i: (i, 0))
    txspec = pl.BlockSpec((NC, R, D), lambda i: (0, i, 0))
    batspec = pl.BlockSpec((1, 1, R), lambda i: (i, 0, 0))
    wspec = pl.BlockSpec((D, D), lambda i: (0, 0))
    bspec = pl.BlockSpec((1, D), lambda i: (0, 0))
    fspec = pl.BlockSpec((D, 1), lambda i: (0, 0))
    return pl.pallas_call(
        body,
        grid=(grid,),
        in_specs=[blk, txspec, batspec, wspec, wspec, bspec,
                  blk, txspec, batspec, wspec, wspec, bspec,
                  pl.BlockSpec((NC, R, D), lambda i: (0, i, 0)),
                  pl.BlockSpec((NC, R, D), lambda i: (0, i, 0)),
                  fspec, fspec, pl.BlockSpec((1, 1), lambda i: (0, 0))],
        out_specs=pl.BlockSpec((G, 1), lambda i: (0, 0)),
        out_shape=jax.ShapeDtypeStruct((G, 1), F32),
        scratch_shapes=[pltpu.VMEM((G, D), F32), pltpu.VMEM((G, D), F32)],
    )


def _pad_edges(edge_index, K0, K1, KM, pad_node):
    E = edge_index.shape[1]
    n0 = NS * K0 * CH
    n1 = NS * K1 * CH
    r = jnp.concatenate(
        [edge_index[0].astype(I32), jnp.full((n0 + n1 - E,), pad_node, I32)])
    c = jnp.concatenate(
        [edge_index[1].astype(I32), jnp.full((n0 + n1 - E,), pad_node, I32)])

    def part(a):
        a0 = a[:n0].reshape(NS, K0, 1, CH)
        a0 = jnp.pad(a0, ((0, 0), (0, KM - K0), (0, 0), (0, 0)),
                     constant_values=pad_node)
        a1 = a[n0:].reshape(NS, K1, 1, CH)
        a1 = jnp.pad(a1, ((0, 0), (0, KM - K1), (0, 0), (0, 0)),
                     constant_values=pad_node)
        return jnp.concatenate([a0, a1], axis=0)

    rp, cp = part(r), part(c)
    return (jnp.concatenate([rp, cp], axis=2),
            jnp.concatenate([rp, rp], axis=2))


def kernel(x1, x2, edge_index1, edge_index2, x1_batch, x2_batch,
           lin1_w, lin1_b, cheb1_w0, cheb1_w1, cheb1_b,
           lin2_w, lin2_b, cheb2_w0, cheb2_w1, cheb2_b,
           fc2_w, fc2_b):
    NN, D = x1.shape
    E = edge_index1.shape[1]
    G = 64
    R = 1024
    NP = -(-NN // R) * R
    pad_node = NN + 8

    NPA = 10112 if NN <= 10112 else NP
    KTOT = -(-E // (NS * CH))
    K0 = KTOT - int(KTOT * 0.33)
    K1 = KTOT - K0
    KM = max(K0, K1)

    x1p = jnp.pad(x1, ((0, NP - NN), (0, 0)))
    x2p = jnp.pad(x2, ((0, NP - NN), (0, 0)))
    rc1, rr1 = _pad_edges(edge_index1, K0, K1, KM, pad_node)
    rc2, rr2 = _pad_edges(edge_index2, K0, K1, KM, pad_node)
    bat1 = jnp.pad(x1_batch.astype(I32), (0, NP - NN),
                   constant_values=G).reshape(NP // R, 1, R)
    bat2 = jnp.pad(x2_batch.astype(I32), (0, NP - NN),
                   constant_values=G).reshape(NP // R, 1, R)

    def up(a):
        return jnp.pad(a, ((0, 0), (0, NP - NPA), (0, 0)))

    sc_scatter = _make_scatter_kernel(NPA, D, K0, K1, KM)
    ones_tab = jnp.ones((NP, D), F32)
    degp1 = up(sc_scatter(ones_tab, rr1))
    degp2 = up(sc_scatter(ones_tab, rr2))
    h1, g1, h2, g2 = _make_tc_pre(NP, D, NN, R)(
        x1p, x2p, degp1, degp2, lin1_w, lin1_b.reshape(1, D),
        lin2_w, lin2_b.reshape(1, D))
    tx1 = up(sc_scatter(g1, rc1))
    tx2 = up(sc_scatter(g2, rc2))
    pred = _make_tc_post(NP, D, NN, G, R)(
        h1, tx1, bat1, cheb1_w0, cheb1_w1, cheb1_b.reshape(1, D),
        h2, tx2, bat2, cheb2_w0, cheb2_w1, cheb2_b.reshape(1, D),
        degp1, degp2, fc2_w[:D], fc2_w[D:], fc2_b.reshape(1, 1))
    return pred.reshape(-1)


# speedup vs baseline: 1.0336x; 1.0336x over previous
import functools

import jax
import jax.numpy as jnp
from jax import lax
from jax.experimental import pallas as pl
from jax.experimental.pallas import tpu as pltpu
from jax.experimental.pallas import tpu_sc as plsc

NC = 2
NS = 16
TILES = NC * NS
CH = 88

F32 = jnp.float32
I32 = jnp.int32


def _sc_mesh():
    return plsc.VectorSubcoreMesh(core_axis_name="c", subcore_axis_name="s")


def _make_scatter_kernel(NPA, D, K0, K1, KM):
    rows_per_tile = NPA // NS

    NBUF = 4
    NI = 6
    S = 2

    def body(g_hbm, idx_hbm, out_hbm, idx_v, rows_v, acc_sh, gsem, ssem,
             isem):
        cid = lax.axis_index("c")
        sid = lax.axis_index("s")
        tid = cid * NS + sid
        KT = jnp.where(cid == 0, K0, K1)
        zeros = jnp.zeros((16,), F32)

        def zbody(i, _):
            for l in range(8):
                rows_v[0, i, pl.ds(l * 16, 16)] = zeros
            return _

        lax.fori_loop(0, CH, zbody, None)
        base = sid * rows_per_tile
        nfull = rows_per_tile // CH
        for r in range(nfull):
            pltpu.sync_copy(rows_v.at[0], acc_sh.at[pl.ds(base + r * CH, CH)])
        rem = rows_per_tile - nfull * CH
        if rem:
            pltpu.sync_copy(rows_v.at[0, pl.ds(0, rem)],
                            acc_sh.at[pl.ds(base + nfull * CH, rem)])
        plsc.subcore_barrier()

        def idx_load(c):
            pltpu.async_copy(idx_hbm.at[tid, c], idx_v.at[lax.rem(c, NI)],
                             isem)

        def idx_wait():
            pltpu.make_async_copy(idx_hbm.at[tid, 0], idx_v.at[0],
                                  isem).wait()

        def gather(c):
            pltpu.async_copy(g_hbm.at[idx_v.at[lax.rem(c, NI), 0]],
                             rows_v.at[lax.rem(c, NBUF)], gsem)

        for c in range(NI):
            idx_load(c)
        for c in range(NBUF):
            idx_wait()
            gather(c)

        def chunk(j, _):
            b = lax.rem(j, NBUF)
            pltpu.make_async_copy(
                g_hbm.at[idx_v.at[0, 0]], rows_v.at[b], gsem).wait()
            pltpu.async_copy(rows_v.at[b],
                             acc_sh.at[idx_v.at[lax.rem(j, NI), 1]],
                             ssem, add=True)

            @pl.when(j >= S)
            def _():
                jd = j - S
                pltpu.make_async_copy(
                    rows_v.at[lax.rem(jd, NBUF)], acc_sh.at[idx_v.at[0, 1]],
                    ssem).wait()

                @pl.when(jd + NI < KT)
                def _():
                    idx_load(jd + NI)

                @pl.when(jd + NBUF < KT)
                def _():
                    idx_wait()
                    gather(jd + NBUF)

            return _

        lax.fori_loop(0, KT, chunk, None)
        for t in range(S):
            pltpu.make_async_copy(
                rows_v.at[t], acc_sh.at[idx_v.at[0, 1]], ssem).wait()

        plsc.subcore_barrier()
        sl = pl.ds(base, rows_per_tile)
        pltpu.sync_copy(acc_sh.at[sl], out_hbm.at[cid, sl])

    return pl.kernel(
        body,
        out_type=jax.ShapeDtypeStruct((NC, NPA, D), F32),
        mesh=_sc_mesh(),
        scratch_types=[
            pltpu.VMEM((NI, 2, CH), I32),
            pltpu.VMEM((NBUF, CH, D), F32),
            pltpu.VMEM_SHARED((NPA, D), F32),
            pltpu.SemaphoreType.DMA,
            pltpu.SemaphoreType.DMA,
            pltpu.SemaphoreType.DMA,
        ],
    )


def _dinv_block(degp, b, i, R, NN, D):
    deg = degp[0, :, 0] + degp[1, :, 0]
    rowid = lax.broadcasted_iota(I32, (R,), 0) + i * R
    valid = (deg > 0) & (rowid < NN)
    return jnp.where(valid, lax.rsqrt(jnp.maximum(deg, 1.0)), 0.0)


def _make_tc_pre(NP, D, NN, R):
    grid = NP // R

    def body(x1_ref, x2_ref, degp1_ref, degp2_ref, w1_ref, b1_ref,
             w2_ref, b2_ref, h1_ref, g1_ref, h2_ref, g2_ref):
        i = pl.program_id(0)
        for b, (x_ref, degp_ref, w_ref, bb_ref, h_ref, g_ref) in enumerate(
                ((x1_ref, degp1_ref, w1_ref, b1_ref, h1_ref, g1_ref),
                 (x2_ref, degp2_ref, w2_ref, b2_ref, h2_ref, g2_ref))):
            h = jnp.maximum(
                jnp.dot(x_ref[...], w_ref[...],
                        preferred_element_type=F32) + bb_ref[...], 0.0)
            dinv = _dinv_block(degp_ref, b, i, R, NN, D)
            h_ref[...] = h
            g_ref[...] = h * dinv[:, None]

    blk = pl.BlockSpec((R, D), lambda i: (i, 0))
    wspec = pl.BlockSpec((D, D), lambda i: (0, 0))
    bspec = pl.BlockSpec((1, D), lambda i: (0, 0))
    return pl.pallas_call(
        body,
        grid=(grid,),
        in_specs=[blk, blk,
                  pl.BlockSpec((NC, R, D), lambda i: (0, i, 0)),
                  pl.BlockSpec((NC, R, D), lambda i: (0, i, 0)),
                  wspec, bspec, wspec, bspec],
        out_specs=[blk, blk, blk, blk],
        out_shape=[jax.ShapeDtypeStruct((NP, D), F32)] * 4,
    )


def _make_tc_post(NP, D, NN, G, R):
    grid = NP // R

    def body(h1_ref, tx1_ref, bat1_ref, w01_ref, w11_ref, bb1_ref,
             h2_ref, tx2_ref, bat2_ref, w02_ref, w12_ref, bb2_ref,
             degp1_ref, degp2_ref, f1_ref, f2_ref, fb_ref,
             pred_ref, p1_ref, p2_ref):
        i = pl.program_id(0)

        @pl.when(i == 0)
        def _():
            p1_ref[...] = jnp.zeros((G, D), F32)
            p2_ref[...] = jnp.zeros((G, D), F32)

        for b, (h_ref, tx_ref, bat_ref, degp_ref, w0_ref, w1_ref, bb_ref,
                p_ref) in \
                enumerate(((h1_ref, tx1_ref, bat1_ref, degp1_ref, w01_ref,
                            w11_ref, bb1_ref, p1_ref),
                           (h2_ref, tx2_ref, bat2_ref, degp2_ref, w02_ref,
                            w12_ref, bb2_ref, p2_ref))):
            dinv = _dinv_block(degp_ref, b, i, R, NN, D)
            t = (tx_ref[0] + tx_ref[1]) * (-dinv)[:, None]
            out = jnp.maximum(
                jnp.dot(h_ref[...], w0_ref[...], preferred_element_type=F32)
                + jnp.dot(t, w1_ref[...], preferred_element_type=F32)
                + bb_ref[...], 0.0)
            bat = bat_ref[0, 0, :]
            mask = (bat[:, None] ==
                    lax.broadcasted_iota(I32, (1, G), 1)).astype(F32)
            p_ref[...] += lax.dot_general(
                mask, out, (((0,), (0,)), ((), ())),
                preferred_element_type=F32)

        pred_ref[...] = (
            jnp.dot(p1_ref[...], f1_ref[...], preferred_element_type=F32)
            + jnp.dot(p2_ref[...], f2_ref[...], preferred_element_type=F32)
            + fb_ref[...])

    blk = pl.BlockSpec((R, D), lambda i: (i, 0))
    txspec = pl.BlockSpec((NC, R, D), lambda i: (0, i, 0))
    batspec = pl.BlockSpec((1, 1, R), lambda i: (i, 0, 0))
    wspec = pl.BlockSpec((D, D), lambda i: (0, 0))
    bspec = pl.BlockSpec((1, D), lambda i: (0, 0))
    fspec = pl.BlockSpec((D, 1), lambda i: (0, 0))
    return pl.pallas_call(
        body,
        grid=(grid,),
        in_specs=[blk, txspec, batspec, wspec, wspec, bspec,
                  blk, txspec, batspec, wspec, wspec, bspec,
                  pl.BlockSpec((NC, R, D), lambda i: (0, i, 0)),
                  pl.BlockSpec((NC, R, D), lambda i: (0, i, 0)),
                  fspec, fspec, pl.BlockSpec((1, 1), lambda i: (0, 0))],
        out_specs=pl.BlockSpec((G, 1), lambda i: (0, 0)),
        out_shape=jax.ShapeDtypeStruct((G, 1), F32),
        scratch_shapes=[pltpu.VMEM((G, D), F32), pltpu.VMEM((G, D), F32)],
    )


def _pad_edges(edge_index, K0, K1, KM, pad_node):
    E = edge_index.shape[1]
    n0 = NS * K0 * CH
    n1 = NS * K1 * CH
    r = jnp.concatenate(
        [edge_index[0].astype(I32), jnp.full((n0 + n1 - E,), pad_node, I32)])
    c = jnp.concatenate(
        [edge_index[1].astype(I32), jnp.full((n0 + n1 - E,), pad_node, I32)])

    def part(a):
        a0 = a[:n0].reshape(NS, K0, 1, CH)
        a0 = jnp.pad(a0, ((0, 0), (0, KM - K0), (0, 0), (0, 0)),
                     constant_values=pad_node)
        a1 = a[n0:].reshape(NS, K1, 1, CH)
        a1 = jnp.pad(a1, ((0, 0), (0, KM - K1), (0, 0), (0, 0)),
                     constant_values=pad_node)
        return jnp.concatenate([a0, a1], axis=0)

    rp, cp = part(r), part(c)
    return (jnp.concatenate([rp, cp], axis=2),
            jnp.concatenate([rp, rp], axis=2))


def kernel(x1, x2, edge_index1, edge_index2, x1_batch, x2_batch,
           lin1_w, lin1_b, cheb1_w0, cheb1_w1, cheb1_b,
           lin2_w, lin2_b, cheb2_w0, cheb2_w1, cheb2_b,
           fc2_w, fc2_b):
    NN, D = x1.shape
    E = edge_index1.shape[1]
    G = 64
    R = 1024
    NP = -(-NN // R) * R
    pad_node = NN + 8

    NPA = 10112 if NN <= 10112 else NP
    KTOT = -(-E // (NS * CH))
    K0 = KTOT - int(KTOT * 0.36)
    K1 = KTOT - K0
    KM = max(K0, K1)

    x1p = jnp.pad(x1, ((0, NP - NN), (0, 0)))
    x2p = jnp.pad(x2, ((0, NP - NN), (0, 0)))
    rc1, rr1 = _pad_edges(edge_index1, K0, K1, KM, pad_node)
    rc2, rr2 = _pad_edges(edge_index2, K0, K1, KM, pad_node)
    bat1 = jnp.pad(x1_batch.astype(I32), (0, NP - NN),
                   constant_values=G).reshape(NP // R, 1, R)
    bat2 = jnp.pad(x2_batch.astype(I32), (0, NP - NN),
                   constant_values=G).reshape(NP // R, 1, R)

    def up(a):
        return jnp.pad(a, ((0, 0), (0, NP - NPA), (0, 0)))

    sc_scatter = _make_scatter_kernel(NPA, D, K0, K1, KM)
    ones_tab = jnp.ones((NP, D), F32)
    degp1 = up(sc_scatter(ones_tab, rr1))
    degp2 = up(sc_scatter(ones_tab, rr2))
    h1, g1, h2, g2 = _make_tc_pre(NP, D, NN, R)(
        x1p, x2p, degp1, degp2, lin1_w, lin1_b.reshape(1, D),
        lin2_w, lin2_b.reshape(1, D))
    tx1 = up(sc_scatter(g1, rc1))
    tx2 = up(sc_scatter(g2, rc2))
    pred = _make_tc_post(NP, D, NN, G, R)(
        h1, tx1, bat1, cheb1_w0, cheb1_w1, cheb1_b.reshape(1, D),
        h2, tx2, bat2, cheb2_w0, cheb2_w1, cheb2_b.reshape(1, D),
        degp1, degp2, fc2_w[:D], fc2_w[D:], fc2_b.reshape(1, 1))
    return pred.reshape(-1)
